# baseline (speedup 1.0000x reference)
import functools

import jax
import jax.numpy as jnp
from jax import lax
from jax.experimental import pallas as pl
from jax.experimental.pallas import tpu as pltpu
from jax.experimental.pallas import tpu_sc as plsc

NC, NS = 2, 16
NW = NC * NS
B, L, D = 4096, 200, 32
CH = 128
LP = 2 * CH
BPW = B // NW
LANES = 16


def _build(interpret=False):
    mesh = plsc.VectorSubcoreMesh(
        core_axis_name="c", subcore_axis_name="s",
        num_cores=NC, num_subcores=NS)

    @functools.partial(
        pl.kernel,
        out_type=jax.ShapeDtypeStruct((NW, BPW, D), jnp.float32),
        mesh=mesh,
        scratch_types=[
            pltpu.VMEM((2 * BPW, CH), jnp.int32),
            pltpu.VMEM((CH, D), jnp.float32),
            pltpu.VMEM((CH, D), jnp.float32),
            pltpu.VMEM((BPW, D), jnp.float32),
            pltpu.SemaphoreType.DMA,
            pltpu.SemaphoreType.DMA,
        ],
        compiler_params=pltpu.CompilerParams(use_tc_tiling_on_sc=False),
        interpret=interpret,
    )
    def set_embed(x_hbm, table_hbm, out_hbm, idx_v, rows0, rows1, out_v,
                  sem0, sem1):
        wid = lax.axis_index("c") * NS + lax.axis_index("s")
        pltpu.sync_copy(x_hbm.at[wid], idx_v)

        def gather_start(c, rows, sem):
            pltpu.async_copy(table_hbm.at[idx_v.at[c]], rows, sem)

        def gather_wait(c, rows, sem):
            pltpu.make_async_copy(table_hbm.at[idx_v.at[c]], rows, sem).wait()

        def reduce_chunk(rows, carry_in):
            def body(t, carry):
                v0, v1 = carry
                for u in range(8):
                    j = t * 8 + u
                    v0 = jnp.maximum(v0, rows[j, pl.ds(0, LANES)])
                    v1 = jnp.maximum(v1, rows[j, pl.ds(LANES, LANES)])
                return v0, v1
            return lax.fori_loop(0, CH // 8, body, carry_in)

        gather_start(0, rows0, sem0)
        neg = jnp.full((LANES,), -jnp.inf, jnp.float32)

        def outer(r, _):
            c0 = r * 2
            gather_start(c0 + 1, rows1, sem1)
            gather_wait(c0, rows0, sem0)
            v = reduce_chunk(rows0, (neg, neg))

            @pl.when(r + 1 < BPW)
            def _prefetch():
                gather_start(c0 + 2, rows0, sem0)

            gather_wait(c0 + 1, rows1, sem1)
            v0, v1 = reduce_chunk(rows1, v)
            out_v[r, pl.ds(0, LANES)] = v0
            out_v[r, pl.ds(LANES, LANES)] = v1
            return 0

        lax.fori_loop(0, BPW, outer, 0)
        pltpu.sync_copy(out_v, out_hbm.at[wid])

    return set_embed


_set_embed = _build()

_TW = 2048


def _tp_body(in_ref, out_ref):
    out_ref[...] = in_ref[...].T


def _tc_transpose(tt):
    n = tt.shape[1]
    return pl.pallas_call(
        _tp_body,
        grid=(pl.cdiv(n, _TW),),
        in_specs=[pl.BlockSpec((D, _TW), lambda i: (0, i))],
        out_specs=pl.BlockSpec((_TW, D), lambda i: (i, 0)),
        out_shape=jax.ShapeDtypeStruct((n, D), jnp.float32),
    )(tt)


def kernel(x, table):
    table_rm = _tc_transpose(jnp.swapaxes(table, 0, 1))
    xp = jnp.concatenate([x, x[:, : LP - L]], axis=1)
    out = _set_embed(xp.reshape(NW, 2 * BPW, CH), table_rm)
    return out.reshape(B, 1, D)


# speedup vs baseline: 1.9352x; 1.9352x over previous
import functools

import jax
import jax.numpy as jnp
from jax import lax
from jax.experimental import pallas as pl
from jax.experimental.pallas import tpu as pltpu
from jax.experimental.pallas import tpu_sc as plsc

NC, NS = 2, 16
NW = NC * NS
B, L, D = 4096, 200, 32
CH = 128
LP = 2 * CH
BPW = B // NW
LANES = 16


def _build(interpret=False):
    mesh = plsc.VectorSubcoreMesh(
        core_axis_name="c", subcore_axis_name="s",
        num_cores=NC, num_subcores=NS)

    @functools.partial(
        pl.kernel,
        out_type=jax.ShapeDtypeStruct((NW, BPW, D), jnp.float32),
        mesh=mesh,
        scratch_types=[
            pltpu.VMEM((2 * BPW, CH), jnp.int32),
            pltpu.VMEM((CH, D), jnp.float32),
            pltpu.VMEM((CH, D), jnp.float32),
            pltpu.VMEM((BPW, D), jnp.float32),
            pltpu.SemaphoreType.DMA,
            pltpu.SemaphoreType.DMA,
        ],
        compiler_params=pltpu.CompilerParams(use_tc_tiling_on_sc=False),
        interpret=interpret,
    )
    def set_embed(x_hbm, table_hbm, out_hbm, idx_v, rows0, rows1, out_v,
                  sem0, sem1):
        wid = lax.axis_index("c") * NS + lax.axis_index("s")
        pltpu.sync_copy(x_hbm.at[wid], idx_v)

        def gather_start(c, rows, sem):
            pltpu.async_copy(table_hbm.at[idx_v.at[c]], rows, sem)

        def gather_wait(c, rows, sem):
            pltpu.make_async_copy(table_hbm.at[idx_v.at[c]], rows, sem).wait()

        def reduce_chunk(rows, carry_in):
            def body(t, carry):
                v0, v1 = carry
                for u in range(8):
                    j = t * 8 + u
                    v0 = jnp.maximum(v0, rows[j, pl.ds(0, LANES)])
                    v1 = jnp.maximum(v1, rows[j, pl.ds(LANES, LANES)])
                return v0, v1
            return lax.fori_loop(0, CH // 8, body, carry_in)

        gather_start(0, rows0, sem0)
        neg = jnp.full((LANES,), -jnp.inf, jnp.float32)

        def outer(r, _):
            c0 = r * 2
            gather_start(c0 + 1, rows1, sem1)
            gather_wait(c0, rows0, sem0)
            v = reduce_chunk(rows0, (neg, neg))

            @pl.when(r + 1 < BPW)
            def _prefetch():
                gather_start(c0 + 2, rows0, sem0)

            gather_wait(c0 + 1, rows1, sem1)
            v0, v1 = reduce_chunk(rows1, v)
            out_v[r, pl.ds(0, LANES)] = v0
            out_v[r, pl.ds(LANES, LANES)] = v1
            return 0

        lax.fori_loop(0, BPW, outer, 0)
        pltpu.sync_copy(out_v, out_hbm.at[wid])

    return set_embed


_set_embed = _build()

_TW = 2048


_G = _TW // 4


def _tp_body(in_ref, out_ref):
    x = in_ref[...]
    slab = jnp.concatenate(
        [x[:, k * _G:(k + 1) * _G] for k in range(4)], axis=0)
    out_ref[...] = slab.T


def _tc_transpose(tt):
    n = tt.shape[1]
    nb = pl.cdiv(n, _TW)
    return pl.pallas_call(
        _tp_body,
        grid=(nb,),
        in_specs=[pl.BlockSpec((D, _TW), lambda i: (0, i))],
        out_specs=pl.BlockSpec((_G, 128), lambda i: (i, 0)),
        out_shape=jax.ShapeDtypeStruct((nb * _G, 128), jnp.float32),
    )(tt)


def kernel(x, table):
    t2 = _tc_transpose(jnp.swapaxes(table, 0, 1))
    table_rm = t2.reshape(t2.shape[0] * 4, D)
    xp = jnp.concatenate([x, x[:, : LP - L]], axis=1)
    xp = (xp & ~2047) + ((xp & 511) << 2) + ((xp >> 9) & 3)
    out = _set_embed(xp.reshape(NW, 2 * BPW, CH), table_rm)
    return out.reshape(B, 1, D)


# speedup vs baseline: 3.1261x; 1.6154x over previous
import functools

import jax
import jax.numpy as jnp
from jax import lax
from jax.experimental import pallas as pl
from jax.experimental.pallas import tpu as pltpu
from jax.experimental.pallas import tpu_sc as plsc

NC, NS = 2, 16
NW = NC * NS
B, L, D = 4096, 200, 32
CH = 128
LP = 2 * CH
BPW = B // NW
LANES = 16


def _build(interpret=False):
    mesh = plsc.VectorSubcoreMesh(
        core_axis_name="c", subcore_axis_name="s",
        num_cores=NC, num_subcores=NS)

    @functools.partial(
        pl.kernel,
        out_type=jax.ShapeDtypeStruct((NW, BPW, D), jnp.float32),
        mesh=mesh,
        scratch_types=[
            pltpu.VMEM((2 * BPW, CH), jnp.int32),
            pltpu.VMEM((CH, D), jnp.float32),
            pltpu.VMEM((CH, D), jnp.float32),
            pltpu.VMEM((BPW, D), jnp.float32),
            pltpu.SemaphoreType.DMA,
            pltpu.SemaphoreType.DMA,
        ],
        compiler_params=pltpu.CompilerParams(use_tc_tiling_on_sc=False),
        interpret=interpret,
    )
    def set_embed(x_hbm, table_hbm, out_hbm, idx_v, rows0, rows1, out_v,
                  sem0, sem1):
        wid = lax.axis_index("c") * NS + lax.axis_index("s")
        pltpu.sync_copy(x_hbm.at[wid], idx_v)

        def gather_start(c, rows, sem):
            pltpu.async_copy(table_hbm.at[idx_v.at[c]], rows, sem)

        def gather_wait(c, rows, sem):
            pltpu.make_async_copy(table_hbm.at[idx_v.at[c]], rows, sem).wait()

        def reduce_chunk(rows, carry_in):
            def body(t, carry):
                v0, v1 = carry
                for u in range(8):
                    j = t * 8 + u
                    v0 = jnp.maximum(v0, rows[j, pl.ds(0, LANES)])
                    v1 = jnp.maximum(v1, rows[j, pl.ds(LANES, LANES)])
                return v0, v1
            return lax.fori_loop(0, CH // 8, body, carry_in)

        gather_start(0, rows0, sem0)
        neg = jnp.full((LANES,), -jnp.inf, jnp.float32)

        def outer(r, _):
            c0 = r * 2
            gather_start(c0 + 1, rows1, sem1)
            gather_wait(c0, rows0, sem0)
            v = reduce_chunk(rows0, (neg, neg))

            @pl.when(r + 1 < BPW)
            def _prefetch():
                gather_start(c0 + 2, rows0, sem0)

            gather_wait(c0 + 1, rows1, sem1)
            v0, v1 = reduce_chunk(rows1, v)
            out_v[r, pl.ds(0, LANES)] = v0
            out_v[r, pl.ds(LANES, LANES)] = v1
            return 0

        lax.fori_loop(0, BPW, outer, 0)
        pltpu.sync_copy(out_v, out_hbm.at[wid])

    return set_embed


_set_embed = _build()

_TW = 8192


_G = _TW // 4


def _tp_body(in_ref, out_ref):
    x = in_ref[...]
    slab = jnp.concatenate(
        [x[:, k * _G:(k + 1) * _G] for k in range(4)], axis=0)
    out_ref[...] = slab.T


def _tc_transpose(tt):
    n = tt.shape[1]
    nb = pl.cdiv(n, _TW)
    return pl.pallas_call(
        _tp_body,
        grid=(nb,),
        in_specs=[pl.BlockSpec((D, _TW), lambda i: (0, i))],
        out_specs=pl.BlockSpec((_G, 128), lambda i: (i, 0)),
        out_shape=jax.ShapeDtypeStruct((nb * _G, 128), jnp.float32),
    )(tt)


def kernel(x, table):
    t2 = _tc_transpose(jnp.swapaxes(table, 0, 1))
    table_rm = t2.reshape(t2.shape[0] * 4, D)
    xp = jnp.concatenate([x, x[:, : LP - L]], axis=1)
    gl = _G.bit_length() - 1
    xp = (xp & ~(_TW - 1)) + ((xp & (_G - 1)) << 2) + ((xp >> gl) & 3)
    out = _set_embed(xp.reshape(NW, 2 * BPW, CH), table_rm)
    return out.reshape(B, 1, D)


# speedup vs baseline: 3.5897x; 1.1483x over previous
import functools

import jax
import jax.numpy as jnp
from jax import lax
from jax.experimental import pallas as pl
from jax.experimental.pallas import tpu as pltpu
from jax.experimental.pallas import tpu_sc as plsc

NC, NS = 2, 16
NW = NC * NS
B, L, D = 4096, 200, 32
CH = 128
LP = 2 * CH
BPW = B // NW
LANES = 16


def _build(interpret=False):
    mesh = plsc.VectorSubcoreMesh(
        core_axis_name="c", subcore_axis_name="s",
        num_cores=NC, num_subcores=NS)

    @functools.partial(
        pl.kernel,
        out_type=jax.ShapeDtypeStruct((NW, BPW, D), jnp.float32),
        mesh=mesh,
        scratch_types=[
            pltpu.VMEM((2 * BPW, CH), jnp.int32),
            pltpu.VMEM((CH, D), jnp.float32),
            pltpu.VMEM((CH, D), jnp.float32),
            pltpu.VMEM((BPW, D), jnp.float32),
            pltpu.SemaphoreType.DMA,
            pltpu.SemaphoreType.DMA,
        ],
        compiler_params=pltpu.CompilerParams(use_tc_tiling_on_sc=False),
        interpret=interpret,
    )
    def set_embed(x_hbm, table_hbm, out_hbm, idx_v, rows0, rows1, out_v,
                  sem0, sem1):
        wid = lax.axis_index("c") * NS + lax.axis_index("s")
        pltpu.sync_copy(x_hbm.at[wid], idx_v)

        def gather_start(c, rows, sem):
            pltpu.async_copy(table_hbm.at[idx_v.at[c]], rows, sem)

        def gather_wait(c, rows, sem):
            pltpu.make_async_copy(table_hbm.at[idx_v.at[c]], rows, sem).wait()

        def reduce_chunk(rows, carry_in):
            def body(t, carry):
                v0, v1 = carry
                for u in range(8):
                    j = t * 8 + u
                    v0 = jnp.maximum(v0, rows[j, pl.ds(0, LANES)])
                    v1 = jnp.maximum(v1, rows[j, pl.ds(LANES, LANES)])
                return v0, v1
            return lax.fori_loop(0, CH // 8, body, carry_in)

        gather_start(0, rows0, sem0)
        neg = jnp.full((LANES,), -jnp.inf, jnp.float32)

        def outer(r, _):
            c0 = r * 2
            gather_start(c0 + 1, rows1, sem1)
            gather_wait(c0, rows0, sem0)
            v = reduce_chunk(rows0, (neg, neg))

            @pl.when(r + 1 < BPW)
            def _prefetch():
                gather_start(c0 + 2, rows0, sem0)

            gather_wait(c0 + 1, rows1, sem1)
            v0, v1 = reduce_chunk(rows1, v)
            out_v[r, pl.ds(0, LANES)] = v0
            out_v[r, pl.ds(LANES, LANES)] = v1
            return 0

        lax.fori_loop(0, BPW, outer, 0)
        pltpu.sync_copy(out_v, out_hbm.at[wid])

    return set_embed


_set_embed = _build()

_TW = 16384


_G = _TW // 4


def _tp_body(in_ref, out_ref):
    x = in_ref[...]
    slab = jnp.concatenate(
        [x[:, k * _G:(k + 1) * _G] for k in range(4)], axis=0)
    out_ref[...] = slab.T


def _tc_transpose(tt):
    n = tt.shape[1]
    nb = pl.cdiv(n, _TW)
    return pl.pallas_call(
        _tp_body,
        grid=(nb,),
        in_specs=[pl.BlockSpec((D, _TW), lambda i: (0, i))],
        out_specs=pl.BlockSpec((_G, 128), lambda i: (i, 0)),
        out_shape=jax.ShapeDtypeStruct((nb * _G, 128), jnp.float32),
    )(tt)


def kernel(x, table):
    t2 = _tc_transpose(jnp.swapaxes(table, 0, 1))
    table_rm = t2.reshape(t2.shape[0] * 4, D)
    xp = jnp.concatenate([x, x[:, : LP - L]], axis=1)
    gl = _G.bit_length() - 1
    xp = (xp & ~(_TW - 1)) + ((xp & (_G - 1)) << 2) + ((xp >> gl) & 3)
    out = _set_embed(xp.reshape(NW, 2 * BPW, CH), table_rm)
    return out.reshape(B, 1, D)


# speedup vs baseline: 3.8102x; 1.0614x over previous
import functools

import jax
import jax.numpy as jnp
from jax import lax
from jax.experimental import pallas as pl
from jax.experimental.pallas import tpu as pltpu
from jax.experimental.pallas import tpu_sc as plsc

NC, NS = 2, 16
NW = NC * NS
B, L, D = 4096, 200, 32
CH = 128
LP = 2 * CH
BPW = B // NW
LANES = 16


def _build(interpret=False):
    mesh = plsc.VectorSubcoreMesh(
        core_axis_name="c", subcore_axis_name="s",
        num_cores=NC, num_subcores=NS)

    @functools.partial(
        pl.kernel,
        out_type=jax.ShapeDtypeStruct((NW, BPW, D), jnp.float32),
        mesh=mesh,
        scratch_types=[
            pltpu.VMEM((2 * BPW, CH), jnp.int32),
            pltpu.VMEM((CH, D), jnp.float32),
            pltpu.VMEM((CH, D), jnp.float32),
            pltpu.VMEM((BPW, D), jnp.float32),
            pltpu.SemaphoreType.DMA,
            pltpu.SemaphoreType.DMA,
        ],
        compiler_params=pltpu.CompilerParams(use_tc_tiling_on_sc=False),
        interpret=interpret,
    )
    def set_embed(x_hbm, table_hbm, out_hbm, idx_v, rows0, rows1, out_v,
                  sem0, sem1):
        wid = lax.axis_index("c") * NS + lax.axis_index("s")
        pltpu.sync_copy(x_hbm.at[wid], idx_v)

        def gather_start(c, rows, sem):
            pltpu.async_copy(table_hbm.at[idx_v.at[c]], rows, sem)

        def gather_wait(c, rows, sem):
            pltpu.make_async_copy(table_hbm.at[idx_v.at[c]], rows, sem).wait()

        def reduce_chunk(rows, carry_in):
            def body(t, carry):
                v0, v1 = carry
                for u in range(8):
                    j = t * 8 + u
                    v0 = jnp.maximum(v0, rows[j, pl.ds(0, LANES)])
                    v1 = jnp.maximum(v1, rows[j, pl.ds(LANES, LANES)])
                return v0, v1
            return lax.fori_loop(0, CH // 8, body, carry_in)

        gather_start(0, rows0, sem0)
        neg = jnp.full((LANES,), -jnp.inf, jnp.float32)

        def outer(r, _):
            c0 = r * 2
            gather_start(c0 + 1, rows1, sem1)
            gather_wait(c0, rows0, sem0)
            v = reduce_chunk(rows0, (neg, neg))

            @pl.when(r + 1 < BPW)
            def _prefetch():
                gather_start(c0 + 2, rows0, sem0)

            gather_wait(c0 + 1, rows1, sem1)
            v0, v1 = reduce_chunk(rows1, v)
            out_v[r, pl.ds(0, LANES)] = v0
            out_v[r, pl.ds(LANES, LANES)] = v1
            return 0

        lax.fori_loop(0, BPW, outer, 0)
        pltpu.sync_copy(out_v, out_hbm.at[wid])

    return set_embed


_set_embed = _build()

_TW = 32768


_G = _TW // 4


def _tp_body(in_ref, out_ref):
    x = in_ref[...]
    slab = jnp.concatenate(
        [x[:, k * _G:(k + 1) * _G] for k in range(4)], axis=0)
    out_ref[...] = slab.T


def _tc_transpose(tt):
    n = tt.shape[1]
    nb = pl.cdiv(n, _TW)
    return pl.pallas_call(
        _tp_body,
        grid=(nb,),
        in_specs=[pl.BlockSpec((D, _TW), lambda i: (0, i))],
        out_specs=pl.BlockSpec((_G, 128), lambda i: (i, 0)),
        out_shape=jax.ShapeDtypeStruct((nb * _G, 128), jnp.float32),
    )(tt)


def kernel(x, table):
    t2 = _tc_transpose(jnp.swapaxes(table, 0, 1))
    table_rm = t2.reshape(t2.shape[0] * 4, D)
    xp = jnp.concatenate([x, x[:, : LP - L]], axis=1)
    gl = _G.bit_length() - 1
    xp = (xp & ~(_TW - 1)) + ((xp & (_G - 1)) << 2) + ((xp >> gl) & 3)
    out = _set_embed(xp.reshape(NW, 2 * BPW, CH), table_rm)
    return out.reshape(B, 1, D)


# speedup vs baseline: 3.8198x; 1.0025x over previous
import functools

import jax
import jax.numpy as jnp
from jax import lax
from jax.experimental import pallas as pl
from jax.experimental.pallas import tpu as pltpu
from jax.experimental.pallas import tpu_sc as plsc

NC, NS = 2, 16
NW = NC * NS
B, L, D = 4096, 200, 32
CH = 128
LP = 2 * CH
BPW = B // NW
LANES = 16


def _build(interpret=False):
    mesh = plsc.VectorSubcoreMesh(
        core_axis_name="c", subcore_axis_name="s",
        num_cores=NC, num_subcores=NS)

    @functools.partial(
        pl.kernel,
        out_type=jax.ShapeDtypeStruct((NW, BPW, D), jnp.float32),
        mesh=mesh,
        scratch_types=[
            pltpu.VMEM((2 * BPW, CH), jnp.int32),
            pltpu.VMEM((CH, D), jnp.float32),
            pltpu.VMEM((CH, D), jnp.float32),
            pltpu.VMEM((BPW, D), jnp.float32),
            pltpu.SemaphoreType.DMA,
            pltpu.SemaphoreType.DMA,
        ],
        compiler_params=pltpu.CompilerParams(use_tc_tiling_on_sc=False),
        interpret=interpret,
    )
    def set_embed(x_hbm, table_hbm, out_hbm, idx_v, rows0, rows1, out_v,
                  sem0, sem1):
        wid = lax.axis_index("c") * NS + lax.axis_index("s")
        pltpu.sync_copy(x_hbm.at[wid], idx_v)

        def gather_start(c, rows, sem):
            pltpu.async_copy(table_hbm.at[idx_v.at[c]], rows, sem)

        def gather_wait(c, rows, sem):
            pltpu.make_async_copy(table_hbm.at[idx_v.at[c]], rows, sem).wait()

        def reduce_chunk(rows, carry_in):
            def body(t, carry):
                v0, v1 = carry
                for u in range(8):
                    j = t * 8 + u
                    v0 = jnp.maximum(v0, rows[j, pl.ds(0, LANES)])
                    v1 = jnp.maximum(v1, rows[j, pl.ds(LANES, LANES)])
                return v0, v1
            return lax.fori_loop(0, CH // 8, body, carry_in)

        gather_start(0, rows0, sem0)
        neg = jnp.full((LANES,), -jnp.inf, jnp.float32)

        def outer(r, _):
            c0 = r * 2
            gather_start(c0 + 1, rows1, sem1)
            gather_wait(c0, rows0, sem0)
            v = reduce_chunk(rows0, (neg, neg))

            @pl.when(r + 1 < BPW)
            def _prefetch():
                gather_start(c0 + 2, rows0, sem0)

            gather_wait(c0 + 1, rows1, sem1)
            v0, v1 = reduce_chunk(rows1, v)
            out_v[r, pl.ds(0, LANES)] = v0
            out_v[r, pl.ds(LANES, LANES)] = v1
            return 0

        lax.fori_loop(0, BPW, outer, 0)
        pltpu.sync_copy(out_v, out_hbm.at[wid])

    return set_embed


_set_embed = _build()

_TW = 65536


_G = _TW // 4


def _tp_body(in_ref, out_ref):
    x = in_ref[...]
    slab = jnp.concatenate(
        [x[:, k * _G:(k + 1) * _G] for k in range(4)], axis=0)
    out_ref[...] = slab.T


def _tc_transpose(tt):
    n = tt.shape[1]
    nb = pl.cdiv(n, _TW)
    return pl.pallas_call(
        _tp_body,
        grid=(nb,),
        in_specs=[pl.BlockSpec((D, _TW), lambda i: (0, i))],
        out_specs=pl.BlockSpec((_G, 128), lambda i: (i, 0)),
        out_shape=jax.ShapeDtypeStruct((nb * _G, 128), jnp.float32),
    )(tt)


def kernel(x, table):
    t2 = _tc_transpose(jnp.swapaxes(table, 0, 1))
    table_rm = t2.reshape(t2.shape[0] * 4, D)
    xp = jnp.concatenate([x, x[:, : LP - L]], axis=1)
    gl = _G.bit_length() - 1
    xp = (xp & ~(_TW - 1)) + ((xp & (_G - 1)) << 2) + ((xp >> gl) & 3)
    out = _set_embed(xp.reshape(NW, 2 * BPW, CH), table_rm)
    return out.reshape(B, 1, D)
